# baseline (speedup 1.0000x reference)
import jax
import jax.numpy as jnp
from jax.experimental import pallas as pl
from jax.experimental.pallas import tpu as pltpu

_BN = 512
_BK = 256


def _mm_kernel(x_ref, w_ref, o_ref, xb_ref):
    j = pl.program_id(0)
    k = pl.program_id(1)

    @pl.when(j == 0)
    def _():
        xb_ref[:, pl.ds(k * _BK, _BK)] = x_ref[...].astype(jnp.bfloat16)

    xk = xb_ref[:, pl.ds(k * _BK, _BK)]
    wb = w_ref[...].astype(jnp.bfloat16)
    contrib = jax.lax.dot_general(
        xk, wb, (((1,), (1,)), ((), ())),
        preferred_element_type=jnp.float32)

    @pl.when(k == 0)
    def _():
        o_ref[...] = contrib

    @pl.when(k > 0)
    def _():
        o_ref[...] += contrib


def kernel(x, weight):
    M, K = x.shape
    N, _ = weight.shape
    grid = (N // _BN, K // _BK)
    return pl.pallas_call(
        _mm_kernel,
        grid=grid,
        in_specs=[
            pl.BlockSpec((M, _BK), lambda j, k: (0, jnp.where(j == 0, k, 0))),
            pl.BlockSpec((_BN, _BK), lambda j, k: (j, k)),
        ],
        out_specs=pl.BlockSpec((M, _BN), lambda j, k: (0, j)),
        out_shape=jax.ShapeDtypeStruct((M, N), jnp.float32),
        scratch_shapes=[pltpu.VMEM((M, K), jnp.bfloat16)],
    )(x, weight)


# speedup vs baseline: 1.6894x; 1.6894x over previous
import jax
import jax.numpy as jnp
from jax.experimental import pallas as pl
from jax.experimental.pallas import tpu as pltpu

_BN = 256


def _mm_kernel(x_ref, w_ref, o_ref, xb_ref):
    @pl.when(pl.program_id(0) == 0)
    def _():
        xb_ref[...] = x_ref[...].astype(jnp.bfloat16)

    wb = w_ref[...].astype(jnp.bfloat16)
    o_ref[...] = jax.lax.dot_general(
        xb_ref[...], wb, (((1,), (1,)), ((), ())),
        preferred_element_type=jnp.float32)


def kernel(x, weight):
    M, K = x.shape
    N, _ = weight.shape
    return pl.pallas_call(
        _mm_kernel,
        grid=(N // _BN,),
        in_specs=[
            pl.BlockSpec((M, K), lambda j: (0, 0)),
            pl.BlockSpec((_BN, K), lambda j: (j, 0)),
        ],
        out_specs=pl.BlockSpec((M, _BN), lambda j: (0, j)),
        out_shape=jax.ShapeDtypeStruct((M, N), jnp.float32),
        scratch_shapes=[pltpu.VMEM((M, K), jnp.bfloat16)],
    )(x, weight)
